# baseline (speedup 1.0000x reference)
import functools

import jax
import jax.numpy as jnp
from jax import lax
from jax.experimental import pallas as pl
from jax.experimental.pallas import tpu as pltpu
from jax.experimental.pallas import tpu_sc as plsc

_NC = 2
_NS = 16
_NW = _NC * _NS
_L = 16
_CH = 128


def _dist2_sc(uv_flat, tab_flat, n_nodes, d):
    n_pairs = uv_flat.shape[0] // 2
    per_w = n_pairs // _NW
    eps_w = 2 * per_w
    nw_gather = d * eps_w
    n_ch = nw_gather // _CH
    n_grp = per_w // _L

    mesh = plsc.VectorSubcoreMesh(core_axis_name="c", subcore_axis_name="s")

    @functools.partial(
        pl.kernel,
        out_type=jax.ShapeDtypeStruct((n_pairs,), jnp.float32),
        mesh=mesh,
        compiler_params=pltpu.CompilerParams(needs_layout_passes=False),
        scratch_types=[
            pltpu.VMEM((eps_w,), jnp.int32),
            pltpu.VMEM((nw_gather,), jnp.int32),
            pltpu.VMEM((nw_gather,), jnp.float32),
            pltpu.VMEM((per_w,), jnp.float32),
            pltpu.SemaphoreType.DMA,
        ],
    )
    def sc_kernel(uv_hbm, tab_hbm, out_hbm, idx_v, gidx_v, cols_v, d2_v, sem):
        wid = lax.axis_index("s") * _NC + lax.axis_index("c")
        pltpu.sync_copy(uv_hbm.at[pl.ds(wid * per_w, per_w)],
                        idx_v.at[pl.ds(0, per_w)])
        pltpu.sync_copy(uv_hbm.at[pl.ds(n_pairs + wid * per_w, per_w)],
                        idx_v.at[pl.ds(per_w, per_w)])

        def gen_body(g, carry):
            iv = idx_v[pl.ds(g * _L, _L)]
            for j in range(d):
                gidx_v[pl.ds(j * eps_w + g * _L, _L)] = iv + j * n_nodes
            return carry

        lax.fori_loop(0, eps_w // _L, gen_body, 0)

        for c in range(n_ch):
            pltpu.async_copy(
                tab_hbm.at[gidx_v.at[pl.ds(c * _CH, _CH)]],
                cols_v.at[pl.ds(c * _CH, _CH)],
                sem,
            )
        pltpu.make_async_copy(
            tab_hbm.at[pl.ds(0, nw_gather)], cols_v, sem).wait()

        def grp_body(g, carry):
            acc = jnp.zeros((_L,), jnp.float32)
            for j in range(d):
                du = cols_v[pl.ds(j * eps_w + g * _L, _L)]
                dv = cols_v[pl.ds(j * eps_w + per_w + g * _L, _L)]
                dd = du - dv
                acc = acc + dd * dd
            d2_v[pl.ds(g * _L, _L)] = acc
            return carry

        lax.fori_loop(0, n_grp, grp_body, 0)
        pltpu.sync_copy(d2_v, out_hbm.at[pl.ds(wid * per_w, per_w)])

    return sc_kernel(uv_flat, tab_flat)


def _loss_tc(d2_mat, lbl_mat, bg):
    def body(bg_ref, d2_ref, lbl_ref, out_ref):
        beta = bg_ref[0]
        gamma = bg_ref[1]
        dist = jnp.sqrt(d2_ref[:])
        z = beta * dist - gamma
        s = jnp.where(lbl_ref[:] == 1, jnp.float32(1.0), jnp.float32(-1.0))
        out_ref[:] = jnp.logaddexp(jnp.float32(0.0), s * z)

    return pl.pallas_call(
        body,
        out_shape=jax.ShapeDtypeStruct(d2_mat.shape, jnp.float32),
        in_specs=[
            pl.BlockSpec(memory_space=pltpu.SMEM),
            pl.BlockSpec(memory_space=pltpu.VMEM),
            pl.BlockSpec(memory_space=pltpu.VMEM),
        ],
        out_specs=pl.BlockSpec(memory_space=pltpu.VMEM),
    )(bg, d2_mat, lbl_mat)


def kernel(pairs, labels, table, beta, gamma):
    n_pairs, _ = pairs.shape
    n_nodes, d = table.shape
    uv_flat = pairs.T.reshape(-1)
    tab_flat = table.T.reshape(-1)
    d2 = _dist2_sc(uv_flat, tab_flat, n_nodes, d)
    rows = n_pairs // 128
    bg = jnp.stack([jnp.asarray(beta, jnp.float32),
                    jnp.asarray(gamma, jnp.float32)])
    loss = _loss_tc(d2.reshape(rows, 128), labels.reshape(rows, 128), bg)
    return loss.reshape(n_pairs)


# speedup vs baseline: 2.7883x; 2.7883x over previous
import functools

import jax
import jax.numpy as jnp
from jax import lax
from jax.experimental import pallas as pl
from jax.experimental.pallas import tpu as pltpu
from jax.experimental.pallas import tpu_sc as plsc

_NC = 2
_NS = 16
_NW = _NC * _NS
_L = 16
_CH = 128


def _dist2_sc(pairs_flat, table):
    n_pairs = pairs_flat.shape[0] // 2
    d = table.shape[1]
    per_w = n_pairs // _NW
    rows_per_w = 2 * per_w
    n_ch = rows_per_w // _CH

    mesh = plsc.VectorSubcoreMesh(core_axis_name="c", subcore_axis_name="s")

    @functools.partial(
        pl.kernel,
        out_type=jax.ShapeDtypeStruct((n_pairs,), jnp.float32),
        mesh=mesh,
        compiler_params=pltpu.CompilerParams(
            needs_layout_passes=False, use_tc_tiling_on_sc=False),
        scratch_types=[
            pltpu.VMEM((rows_per_w,), jnp.int32),
            pltpu.VMEM((rows_per_w, d), jnp.float32),
            pltpu.VMEM((per_w,), jnp.float32),
            pltpu.SemaphoreType.DMA((8,)),
        ],
    )
    def sc_kernel(pairs_hbm, table_hbm, out_hbm, idx_v, rows_v, d2_v, sems):
        wid = lax.axis_index("s") * _NC + lax.axis_index("c")
        pltpu.sync_copy(pairs_hbm.at[pl.ds(wid * rows_per_w, rows_per_w)],
                        idx_v)
        copies = [
            pltpu.async_copy(
                table_hbm.at[idx_v.at[pl.ds(c * _CH, _CH)]],
                rows_v.at[pl.ds(c * _CH, _CH), :],
                sems.at[c],
            )
            for c in range(n_ch)
        ]

        lanes = lax.iota(jnp.int32, _L)
        m_last = lanes == (_L - 1)
        pairs_per_ch = _CH // 2

        for c in range(n_ch):
            copies[c].wait()

            def pair_body(i, carry):
                dd = rows_v[2 * i, :] - rows_v[2 * i + 1, :]
                cs = plsc.cumsum(dd * dd)
                plsc.store_scatter(d2_v, [jnp.full((_L,), i, jnp.int32)],
                                   cs, mask=m_last)
                return carry

            lax.fori_loop(c * pairs_per_ch, (c + 1) * pairs_per_ch,
                          pair_body, 0)
        pltpu.sync_copy(d2_v, out_hbm.at[pl.ds(wid * per_w, per_w)])

    return sc_kernel(pairs_flat, table)


def _loss_tc(d2_mat, lbl_mat, bg):
    def body(bg_ref, d2_ref, lbl_ref, out_ref):
        beta = bg_ref[0]
        gamma = bg_ref[1]
        dist = jnp.sqrt(d2_ref[:])
        z = beta * dist - gamma
        s = jnp.where(lbl_ref[:] == 1, jnp.float32(1.0), jnp.float32(-1.0))
        out_ref[:] = jnp.logaddexp(jnp.float32(0.0), s * z)

    return pl.pallas_call(
        body,
        out_shape=jax.ShapeDtypeStruct(d2_mat.shape, jnp.float32),
        in_specs=[
            pl.BlockSpec(memory_space=pltpu.SMEM),
            pl.BlockSpec(memory_space=pltpu.VMEM),
            pl.BlockSpec(memory_space=pltpu.VMEM),
        ],
        out_specs=pl.BlockSpec(memory_space=pltpu.VMEM),
    )(bg, d2_mat, lbl_mat)


def kernel(pairs, labels, table, beta, gamma):
    n_pairs = pairs.shape[0]
    d2 = _dist2_sc(pairs.reshape(-1), table)
    rows = n_pairs // 128
    bg = jnp.stack([jnp.asarray(beta, jnp.float32),
                    jnp.asarray(gamma, jnp.float32)])
    loss = _loss_tc(d2.reshape(rows, 128), labels.reshape(rows, 128), bg)
    return loss.reshape(n_pairs)


# speedup vs baseline: 3.7204x; 1.3343x over previous
import functools

import jax
import jax.numpy as jnp
from jax import lax
from jax.experimental import pallas as pl
from jax.experimental.pallas import tpu as pltpu
from jax.experimental.pallas import tpu_sc as plsc

_NC = 2
_NS = 16
_NW = _NC * _NS
_L = 16


def _dist2_sc(pairs_flat, table):
    n_pairs = pairs_flat.shape[0] // 2
    v_nodes, d = table.shape
    per_w = n_pairs // _NW
    eps_w = 2 * per_w
    grp = 8

    mesh = plsc.VectorSubcoreMesh(core_axis_name="c", subcore_axis_name="s")

    @functools.partial(
        pl.kernel,
        out_type=jax.ShapeDtypeStruct((n_pairs,), jnp.float32),
        mesh=mesh,
        compiler_params=pltpu.CompilerParams(
            needs_layout_passes=False, use_tc_tiling_on_sc=True),
        scratch_types=[
            pltpu.VMEM((eps_w,), jnp.int32),
            pltpu.SMEM((eps_w,), jnp.int32),
            pltpu.VMEM((2, 32, grp, d), jnp.float32),
            pltpu.VMEM((eps_w * d,), jnp.float32),
            pltpu.VMEM((per_w,), jnp.float32),
            pltpu.SemaphoreType.DMA((2,)),
        ],
    )
    def sc_kernel(pairs_hbm, table_hbm, out_hbm, idx_v, idx_s, tiles_v,
                  rows_v, d2_v, sems):
        wid = lax.axis_index("s") * _NC + lax.axis_index("c")
        tab3 = table_hbm.reshape(v_nodes // grp, grp, d)
        pltpu.sync_copy(pairs_hbm.at[pl.ds(wid * eps_w, eps_w)], idx_v)

        lanes0 = lax.iota(jnp.int32, _L)

        def to_smem(g, carry):
            iv = idx_v[pl.ds(g * _L, _L)]
            for j in range(_L):
                s = jnp.max(jnp.where(lanes0 == j, iv, jnp.int32(-1)))
                idx_s[g * _L + j] = s
            return carry

        lax.fori_loop(0, eps_w // _L, to_smem, 0)

        tb = 32
        n_rounds = eps_w // tb

        def fire_round(r, b):
            def fire(e, carry):
                i = idx_s[e]
                pltpu.async_copy(tab3.at[i // grp],
                                 tiles_v.at[b].at[e - r * tb], sems.at[b])
                return carry
            lax.fori_loop(r * tb, (r + 1) * tb, fire, 0)

        fire_round(0, 0)
        for r in range(n_rounds):
            b = r % 2
            if r + 1 < n_rounds:
                fire_round(r + 1, (r + 1) % 2)
            pltpu.make_async_copy(tab3.at[pl.ds(0, tb)], tiles_v.at[b],
                                  sems.at[b]).wait()

            def extract(e, carry):
                i = idx_s[e]
                rows_v[pl.ds(e * d, d)] = tiles_v[b, e - r * tb, i % grp, :]
                return carry

            lax.fori_loop(r * tb, (r + 1) * tb, extract, 0)

        lanes = lax.iota(jnp.int32, _L)
        m_last = lanes == (_L - 1)

        def pair_body(p, carry):
            dd = (rows_v[pl.ds(2 * p * d, d)]
                  - rows_v[pl.ds((2 * p + 1) * d, d)])
            cs = plsc.cumsum(dd * dd)
            plsc.store_scatter(d2_v, [jnp.full((_L,), p, jnp.int32)],
                               cs, mask=m_last)
            return carry

        lax.fori_loop(0, per_w, pair_body, 0)
        pltpu.sync_copy(d2_v, out_hbm.at[pl.ds(wid * per_w, per_w)])

    return sc_kernel(pairs_flat, table)


def _loss_tc(d2_mat, lbl_mat, bg):
    def body(bg_ref, d2_ref, lbl_ref, out_ref):
        beta = bg_ref[0]
        gamma = bg_ref[1]
        dist = jnp.sqrt(d2_ref[:])
        z = beta * dist - gamma
        s = jnp.where(lbl_ref[:] == 1, jnp.float32(1.0), jnp.float32(-1.0))
        out_ref[:] = jnp.logaddexp(jnp.float32(0.0), s * z)

    return pl.pallas_call(
        body,
        out_shape=jax.ShapeDtypeStruct(d2_mat.shape, jnp.float32),
        in_specs=[
            pl.BlockSpec(memory_space=pltpu.SMEM),
            pl.BlockSpec(memory_space=pltpu.VMEM),
            pl.BlockSpec(memory_space=pltpu.VMEM),
        ],
        out_specs=pl.BlockSpec(memory_space=pltpu.VMEM),
    )(bg, d2_mat, lbl_mat)


def kernel(pairs, labels, table, beta, gamma):
    n_pairs = pairs.shape[0]
    d2 = _dist2_sc(pairs.reshape(-1), table)
    rows = n_pairs // 128
    bg = jnp.stack([jnp.asarray(beta, jnp.float32),
                    jnp.asarray(gamma, jnp.float32)])
    loss = _loss_tc(d2.reshape(rows, 128), labels.reshape(rows, 128), bg)
    return loss.reshape(n_pairs)
